# baseline (speedup 1.0000x reference)
import functools

import jax
import jax.numpy as jnp
from jax import lax
from jax.experimental import pallas as pl
from jax.experimental.pallas import tpu as pltpu
from jax.experimental.pallas import tpu_sc as plsc

DIM = 2048
E = 8
HID = 1024
BM = 256
CHUNK = 16
NBUF = 3
RB = 1024
META_W = 128


_TRIL = None


def _router_meta_body(x_ref, w_ref, tril_ref, dest_ref, meta_ref,
                      am_s, rank_s, cnt_s):
    i = pl.program_id(0)
    nblk = pl.num_programs(0) - 1

    @pl.when(i < nblk)
    def _():
        logits = lax.dot_general(
            x_ref[...], w_ref[...], (((1,), (1,)), ((), ())),
            preferred_element_type=jnp.float32)
        am = jnp.argmax(logits, axis=-1).astype(jnp.int32)
        am_s[i, :] = am
        ohf = (am[:, None] == lax.broadcasted_iota(jnp.int32, (RB, E), 1)
               ).astype(jnp.float32)
        cnt_s[i, :] = jnp.sum(ohf, axis=0).astype(jnp.int32)
        cumf = lax.dot_general(
            tril_ref[...], ohf, (((1,), (0,)), ((), ())),
            preferred_element_type=jnp.float32)
        rank_s[i, :] = (jnp.sum(cumf * ohf, axis=1) - 1.0).astype(jnp.int32)

    @pl.when(i == nblk)
    def _():
        cnt = cnt_s[...]
        cntf = cnt.astype(jnp.float32)
        strict = (lax.broadcasted_iota(jnp.int32, (nblk, nblk), 0)
                  > lax.broadcasted_iota(jnp.int32, (nblk, nblk), 1)
                  ).astype(jnp.float32)
        prior = lax.dot_general(
            strict, cntf, (((0,), (0,)), ((), ())),
            preferred_element_type=jnp.float32)
        total = jnp.sum(cnt, axis=0, keepdims=True)
        pcf = (((total + BM - 1) // BM) * BM).astype(jnp.float32)
        incl = (lax.broadcasted_iota(jnp.int32, (E, E), 0)
                <= lax.broadcasted_iota(jnp.int32, (E, E), 1)
                ).astype(jnp.float32)
        ends = lax.dot_general(
            pcf, incl, (((1,), (0,)), ((), ())),
            preferred_element_type=jnp.float32)
        base = (ends - pcf) + prior
        am_all = am_s[...]
        oh3 = am_all[:, :, None] == lax.broadcasted_iota(
            jnp.int32, (nblk, RB, E), 2)
        dest3 = jnp.sum(jnp.where(oh3, base[:, None, :], 0.0), axis=2
                        ).astype(jnp.int32) + rank_s[...]
        dest_ref[...] = dest3[:, None, :]
        lane = lax.broadcasted_iota(jnp.int32, (1, META_W), 1)
        bstart = (lane * BM).astype(jnp.float32)
        be = jnp.sum((bstart[:, :, None] >= ends[None, :, :]).astype(jnp.int32),
                     axis=2)
        be = jnp.clip(be, 0, E - 1)
        n_active = (ends[0, E - 1] / BM).astype(jnp.int32)
        meta_ref[...] = jnp.where(lane == META_W - 1, n_active, be)


def _router_meta(xf, W_router):
    global _TRIL
    if _TRIL is None:
        import numpy as np
        _TRIL = jnp.asarray(np.tril(np.ones((RB, RB), np.float32)))
    T = xf.shape[0]
    nblk = T // RB
    return pl.pallas_call(
        _router_meta_body,
        grid=(nblk + 1,),
        in_specs=[
            pl.BlockSpec((RB, DIM), lambda i: (jnp.minimum(i, nblk - 1), 0)),
            pl.BlockSpec((E, DIM), lambda i: (0, 0)),
            pl.BlockSpec((RB, RB), lambda i: (0, 0)),
        ],
        out_specs=[
            pl.BlockSpec((nblk, 1, RB), lambda i: (0, 0, 0)),
            pl.BlockSpec((1, META_W), lambda i: (0, 0)),
        ],
        out_shape=[
            jax.ShapeDtypeStruct((nblk, 1, RB), jnp.int32),
            jax.ShapeDtypeStruct((1, META_W), jnp.int32),
        ],
        scratch_shapes=[
            pltpu.VMEM((nblk, RB), jnp.int32),
            pltpu.VMEM((nblk, RB), jnp.int32),
            pltpu.VMEM((nblk, E), jnp.int32),
        ],
    )(xf, W_router, _TRIL)


KH = 2


def _mlp_body(be_ref, na_ref, x_ref, fc_ref, proj_ref, out_ref):
    b = pl.program_id(0)
    k = pl.program_id(1)

    @pl.when(b < na_ref[0])
    def _():
        h = lax.dot_general(
            x_ref[...], fc_ref[0], (((1,), (1,)), ((), ())),
            preferred_element_type=jnp.float32)
        h = jnp.where(h >= 0, h, 0.5 * h)
        h = h * h
        part = lax.dot_general(
            h, proj_ref[0], (((1,), (1,)), ((), ())),
            preferred_element_type=jnp.float32)

        @pl.when(k == 0)
        def _():
            out_ref[...] = part

        @pl.when(k > 0)
        def _():
            out_ref[...] += part


def _grouped_mlp(xg, W_fc, W_proj, block_expert, n_active, nb):
    grid_spec = pltpu.PrefetchScalarGridSpec(
        num_scalar_prefetch=2,
        grid=(nb, KH),
        in_specs=[
            pl.BlockSpec((BM, DIM), lambda b, k, be, na: (b, 0)),
            pl.BlockSpec((1, HID // KH, DIM), lambda b, k, be, na: (be[b], k, 0)),
            pl.BlockSpec((1, DIM, HID // KH), lambda b, k, be, na: (be[b], 0, k)),
        ],
        out_specs=pl.BlockSpec((BM, DIM), lambda b, k, be, na: (b, 0)),
    )
    return pl.pallas_call(
        _mlp_body,
        grid_spec=grid_spec,
        out_shape=jax.ShapeDtypeStruct((xg.shape[0], DIM), jnp.float32),
    )(block_expert, n_active, xg, W_fc, W_proj)


def _sc_scratch(n, D):
    return (
        [pltpu.VMEM((n, CHUNK), jnp.int32)]
        + [pltpu.VMEM((CHUNK, D), jnp.float32) for _ in range(NBUF)]
        + [pltpu.SemaphoreType.DMA for _ in range(2 * NBUF)]
    )


@functools.cache
def _make_sc_scatter_rows(T, PAD, D):
    info = plsc.get_sparse_core_info()
    NC, NS = info.num_cores, info.num_subcores
    NW = NC * NS
    rows_pw = T // NW
    n = rows_pw // CHUNK
    mesh = plsc.VectorSubcoreMesh(core_axis_name="c", subcore_axis_name="s")

    @functools.partial(
        pl.kernel, mesh=mesh,
        out_type=jax.ShapeDtypeStruct((PAD, D), jnp.float32),
        scratch_types=_sc_scratch(n, D),
    )
    def k(x_hbm, idx2_hbm, out_hbm, idx_all, *rest):
        bufs, sin, sout = rest[:NBUF], rest[NBUF:2 * NBUF], rest[2 * NBUF:]
        wid = lax.axis_index("s") * NC + lax.axis_index("c")
        r0 = wid * rows_pw
        pltpu.sync_copy(idx2_hbm.at[pl.ds(wid * n, n)], idx_all)
        ics = [None] * n
        ocs = [None] * n
        for c in range(min(NBUF, n)):
            off = pl.multiple_of(r0 + c * CHUNK, CHUNK)
            ics[c] = pltpu.async_copy(
                x_hbm.at[pl.ds(off, CHUNK)], bufs[c % NBUF], sin[c % NBUF])
        for c in range(n):
            b = c % NBUF
            ics[c].wait()
            ocs[c] = pltpu.async_copy(bufs[b], out_hbm.at[idx_all.at[c]], sout[b])
            nx = c + NBUF
            if nx < n:
                ocs[c].wait()
                off = pl.multiple_of(r0 + nx * CHUNK, CHUNK)
                ics[nx] = pltpu.async_copy(
                    x_hbm.at[pl.ds(off, CHUNK)], bufs[b], sin[b])
        for c in range(max(n - NBUF, 0), n):
            ocs[c].wait()

    return k


@functools.cache
def _make_sc_gather_rows(T, PAD, D):
    info = plsc.get_sparse_core_info()
    NC, NS = info.num_cores, info.num_subcores
    NW = NC * NS
    rows_pw = T // NW
    n = rows_pw // CHUNK
    mesh = plsc.VectorSubcoreMesh(core_axis_name="c", subcore_axis_name="s")

    @functools.partial(
        pl.kernel, mesh=mesh,
        out_type=jax.ShapeDtypeStruct((T, D), jnp.float32),
        scratch_types=_sc_scratch(n, D),
    )
    def k(tbl_hbm, idx2_hbm, out_hbm, idx_all, *rest):
        bufs, sin, sout = rest[:NBUF], rest[NBUF:2 * NBUF], rest[2 * NBUF:]
        wid = lax.axis_index("s") * NC + lax.axis_index("c")
        r0 = wid * rows_pw
        pltpu.sync_copy(idx2_hbm.at[pl.ds(wid * n, n)], idx_all)
        ics = [None] * n
        ocs = [None] * n
        for c in range(min(NBUF, n)):
            ics[c] = pltpu.async_copy(
                tbl_hbm.at[idx_all.at[c]], bufs[c % NBUF], sin[c % NBUF])
        for c in range(n):
            b = c % NBUF
            ics[c].wait()
            off = pl.multiple_of(r0 + c * CHUNK, CHUNK)
            ocs[c] = pltpu.async_copy(bufs[b], out_hbm.at[pl.ds(off, CHUNK)], sout[b])
            nx = c + NBUF
            if nx < n:
                ocs[c].wait()
                ics[nx] = pltpu.async_copy(
                    tbl_hbm.at[idx_all.at[nx]], bufs[b], sin[b])
        for c in range(max(n - NBUF, 0), n):
            ocs[c].wait()

    return k


def kernel(x, W_fc, W_proj, W_router):
    bsz, seqlen, dim = x.shape
    T = bsz * seqlen
    xf = x.reshape(T, dim)
    pad_m = T + E * BM
    nb = pad_m // BM

    dest, meta = _router_meta(xf, W_router)
    dest2 = dest.reshape(T // CHUNK, CHUNK)
    block_expert = meta[0, :nb]
    n_active = meta[0, META_W - 1:META_W]

    xg = _make_sc_scatter_rows(T, pad_m, dim)(xf, dest2)

    yg = _grouped_mlp(xg, W_fc, W_proj, block_expert, n_active, nb)

    out = _make_sc_gather_rows(T, pad_m, dim)(yg, dest2)

    return out.reshape(bsz, seqlen, dim)


# speedup vs baseline: 1.4516x; 1.4516x over previous
import functools

import jax
import jax.numpy as jnp
from jax import lax
from jax.experimental import pallas as pl
from jax.experimental.pallas import tpu as pltpu
from jax.experimental.pallas import tpu_sc as plsc

DIM = 2048
E = 8
HID = 1024
BM = 256
CHUNK = 16
NBUF = 3
RB = 1024
META_W = 128


def _router_meta_body(x_ref, w_ref, dest_ref, meta_ref, idx_s, cnt_s):
    p = pl.program_id(0)
    i = pl.program_id(1)
    nblk = pl.num_programs(1)

    @pl.when(p == 0)
    def _():
        logits = lax.dot_general(
            x_ref[...], w_ref[...], (((1,), (1,)), ((), ())),
            preferred_element_type=jnp.float32)
        am = jnp.argmax(logits, axis=-1).astype(jnp.int32)
        idx_s[i, :] = am
        oh = (am[:, None] == lax.broadcasted_iota(jnp.int32, (RB, E), 1))
        cnt_s[i, :] = jnp.sum(oh.astype(jnp.int32), axis=0)
        dest_ref[...] = am[None, None, :]

    @pl.when(p == 1)
    def _():
        am = idx_s[i, :]
        ohf = (am[:, None] == lax.broadcasted_iota(jnp.int32, (RB, E), 1)
               ).astype(jnp.float32)
        tril = (lax.broadcasted_iota(jnp.int32, (RB, RB), 0)
                >= lax.broadcasted_iota(jnp.int32, (RB, RB), 1)
                ).astype(jnp.float32)
        cumf = lax.dot_general(
            tril, ohf, (((1,), (0,)), ((), ())),
            preferred_element_type=jnp.float32)
        cnt = cnt_s[...]
        row = lax.broadcasted_iota(jnp.int32, (nblk, E), 0)
        prior = jnp.sum(jnp.where(row < i, cnt, 0), axis=0,
                        keepdims=True).astype(jnp.float32)
        total = jnp.sum(cnt, axis=0, keepdims=True)
        pc = ((total + BM - 1) // BM) * BM
        pcf = pc.astype(jnp.float32)
        incl = (lax.broadcasted_iota(jnp.int32, (E, E), 0)
                <= lax.broadcasted_iota(jnp.int32, (E, E), 1)
                ).astype(jnp.float32)
        ends = lax.dot_general(
            pcf, incl, (((1,), (0,)), ((), ())),
            preferred_element_type=jnp.float32)
        base = ends - pcf + prior
        dest_f = (lax.dot_general(
            ohf, base, (((1,), (1,)), ((), ())),
            preferred_element_type=jnp.float32)[:, 0]
            + jnp.sum(cumf * ohf, axis=1) - 1.0)
        dest_ref[...] = dest_f.astype(jnp.int32)[None, None, :]

        @pl.when(i == nblk - 1)
        def _():
            lane = lax.broadcasted_iota(jnp.int32, (1, META_W), 1)
            bstart = (lane * BM).astype(jnp.float32)
            be = jnp.sum((bstart[:, :, None] >= ends[None, :, :]).astype(jnp.int32),
                         axis=2)
            be = jnp.clip(be, 0, E - 1)
            n_active = (ends[0, E - 1] / BM).astype(jnp.int32)
            meta_ref[...] = jnp.where(lane == META_W - 1, n_active, be)


def _router_meta(xf, W_router):
    T = xf.shape[0]
    nblk = T // RB
    return pl.pallas_call(
        _router_meta_body,
        grid=(2, nblk),
        in_specs=[
            pl.BlockSpec((RB, DIM), lambda p, i: (i * (1 - p), 0)),
            pl.BlockSpec((E, DIM), lambda p, i: (0, 0)),
        ],
        out_specs=[
            pl.BlockSpec((1, 1, RB), lambda p, i: (i, 0, 0)),
            pl.BlockSpec((1, META_W), lambda p, i: (0, 0)),
        ],
        out_shape=[
            jax.ShapeDtypeStruct((nblk, 1, RB), jnp.int32),
            jax.ShapeDtypeStruct((1, META_W), jnp.int32),
        ],
        scratch_shapes=[
            pltpu.VMEM((nblk, RB), jnp.int32),
            pltpu.VMEM((nblk, E), jnp.int32),
        ],
    )(xf, W_router)


def _mlp_body(be_ref, na_ref, x_ref, fc_ref, proj_ref, out_ref):
    b = pl.program_id(0)

    @pl.when(b < na_ref[0])
    def _():
        h = lax.dot_general(
            x_ref[...], fc_ref[0], (((1,), (1,)), ((), ())),
            preferred_element_type=jnp.float32)
        h = jnp.where(h >= 0, h, 0.5 * h)
        h = h * h
        out_ref[...] = lax.dot_general(
            h, proj_ref[0], (((1,), (1,)), ((), ())),
            preferred_element_type=jnp.float32)


def _grouped_mlp(xg, W_fc, W_proj, block_expert, n_active, nb):
    grid_spec = pltpu.PrefetchScalarGridSpec(
        num_scalar_prefetch=2,
        grid=(nb,),
        in_specs=[
            pl.BlockSpec((BM, DIM),
                         lambda b, be, na: (jnp.minimum(b, na[0] - 1), 0)),
            pl.BlockSpec((1, HID, DIM),
                         lambda b, be, na: (be[jnp.minimum(b, na[0] - 1)], 0, 0)),
            pl.BlockSpec((1, DIM, HID),
                         lambda b, be, na: (be[jnp.minimum(b, na[0] - 1)], 0, 0)),
        ],
        out_specs=pl.BlockSpec((BM, DIM),
                               lambda b, be, na: (jnp.minimum(b, na[0] - 1), 0)),
    )
    return pl.pallas_call(
        _mlp_body,
        grid_spec=grid_spec,
        out_shape=jax.ShapeDtypeStruct((xg.shape[0], DIM), jnp.float32),
    )(block_expert, n_active, xg, W_fc, W_proj)


def _sc_scratch(n, D):
    return (
        [pltpu.VMEM((n, CHUNK), jnp.int32)]
        + [pltpu.VMEM((CHUNK, D), jnp.float32) for _ in range(NBUF)]
        + [pltpu.SemaphoreType.DMA for _ in range(2 * NBUF)]
    )


@functools.cache
def _make_sc_scatter_rows(T, PAD, D):
    info = plsc.get_sparse_core_info()
    NC, NS = info.num_cores, info.num_subcores
    NW = NC * NS
    rows_pw = T // NW
    n = rows_pw // CHUNK
    mesh = plsc.VectorSubcoreMesh(core_axis_name="c", subcore_axis_name="s")

    @functools.partial(
        pl.kernel, mesh=mesh,
        out_type=jax.ShapeDtypeStruct((PAD, D), jnp.float32),
        scratch_types=_sc_scratch(n, D),
    )
    def k(x_hbm, idx2_hbm, out_hbm, idx_all, *rest):
        bufs, sin, sout = rest[:NBUF], rest[NBUF:2 * NBUF], rest[2 * NBUF:]
        wid = lax.axis_index("s") * NC + lax.axis_index("c")
        r0 = wid * rows_pw
        pltpu.sync_copy(idx2_hbm.at[pl.ds(wid * n, n)], idx_all)
        ics = [None] * n
        ocs = [None] * n
        for c in range(min(NBUF, n)):
            off = pl.multiple_of(r0 + c * CHUNK, CHUNK)
            ics[c] = pltpu.async_copy(
                x_hbm.at[pl.ds(off, CHUNK)], bufs[c % NBUF], sin[c % NBUF])
        for c in range(n):
            b = c % NBUF
            ics[c].wait()
            ocs[c] = pltpu.async_copy(bufs[b], out_hbm.at[idx_all.at[c]], sout[b])
            nx = c + NBUF
            if nx < n:
                ocs[c].wait()
                off = pl.multiple_of(r0 + nx * CHUNK, CHUNK)
                ics[nx] = pltpu.async_copy(
                    x_hbm.at[pl.ds(off, CHUNK)], bufs[b], sin[b])
        for c in range(max(n - NBUF, 0), n):
            ocs[c].wait()

    return k


@functools.cache
def _make_sc_gather_rows(T, PAD, D):
    info = plsc.get_sparse_core_info()
    NC, NS = info.num_cores, info.num_subcores
    NW = NC * NS
    rows_pw = T // NW
    n = rows_pw // CHUNK
    mesh = plsc.VectorSubcoreMesh(core_axis_name="c", subcore_axis_name="s")

    @functools.partial(
        pl.kernel, mesh=mesh,
        out_type=jax.ShapeDtypeStruct((T, D), jnp.float32),
        scratch_types=_sc_scratch(n, D),
    )
    def k(tbl_hbm, idx2_hbm, out_hbm, idx_all, *rest):
        bufs, sin, sout = rest[:NBUF], rest[NBUF:2 * NBUF], rest[2 * NBUF:]
        wid = lax.axis_index("s") * NC + lax.axis_index("c")
        r0 = wid * rows_pw
        pltpu.sync_copy(idx2_hbm.at[pl.ds(wid * n, n)], idx_all)
        ics = [None] * n
        ocs = [None] * n
        for c in range(min(NBUF, n)):
            ics[c] = pltpu.async_copy(
                tbl_hbm.at[idx_all.at[c]], bufs[c % NBUF], sin[c % NBUF])
        for c in range(n):
            b = c % NBUF
            ics[c].wait()
            off = pl.multiple_of(r0 + c * CHUNK, CHUNK)
            ocs[c] = pltpu.async_copy(bufs[b], out_hbm.at[pl.ds(off, CHUNK)], sout[b])
            nx = c + NBUF
            if nx < n:
                ocs[c].wait()
                ics[nx] = pltpu.async_copy(
                    tbl_hbm.at[idx_all.at[nx]], bufs[b], sin[b])
        for c in range(max(n - NBUF, 0), n):
            ocs[c].wait()

    return k


def kernel(x, W_fc, W_proj, W_router):
    bsz, seqlen, dim = x.shape
    T = bsz * seqlen
    xf = x.reshape(T, dim)
    pad_m = T + E * BM
    nb = pad_m // BM

    dest, meta = _router_meta(xf, W_router)
    dest2 = dest.reshape(T // CHUNK, CHUNK)
    block_expert = meta[0, :nb]
    n_active = meta[0, META_W - 1:META_W]

    xg = _make_sc_scatter_rows(T, pad_m, dim)(xf, dest2)

    yg = _grouped_mlp(xg, W_fc, W_proj, block_expert, n_active, nb)

    out = _make_sc_gather_rows(T, pad_m, dim)(yg, dest2)

    return out.reshape(bsz, seqlen, dim)
